# initial kernel scaffold (unmeasured)
import jax
import jax.numpy as jnp
from jax import lax
from jax.experimental import pallas as pl
from jax.experimental.pallas import tpu as pltpu

N_DEV = 8
B = 2
S_LOC = 512
HQ = 8
DH = 64
D_MODEL = 768
D_QK = HQ * DH
S_GLB = N_DEV * S_LOC
BLK = 64
SCALE = 0.125


def kernel(x, Wq, K_ext, V_ext, Wo):
    def body(x_ref, wq_ref, k_ref, v_ref, wo_ref, out_ref,
             kv_ref, send_sems, recv_sems):
        my = lax.axis_index("i")
        left = (my - 1 + N_DEV) % N_DEV
        right = (my + 1) % N_DEV

        barrier_sem = pltpu.get_barrier_semaphore()
        for nbr in (left, right):
            pl.semaphore_signal(barrier_sem, inc=1, device_id=(nbr,),
                                device_id_type=pl.DeviceIdType.MESH)
        pl.semaphore_wait(barrier_sem, 2)

        kv_ref[my, 0] = k_ref[...]
        kv_ref[my, 1] = v_ref[...]

        for h in range(N_DEV - 1):
            src_slot = (my - h + N_DEV) % N_DEV
            rdma = pltpu.make_async_remote_copy(
                src_ref=kv_ref.at[src_slot],
                dst_ref=kv_ref.at[src_slot],
                send_sem=send_sems.at[h],
                recv_sem=recv_sems.at[h],
                device_id=(right,),
                device_id_type=pl.DeviceIdType.MESH,
            )
            rdma.start()
            rdma.wait()

        row_ids = my * S_LOC + lax.broadcasted_iota(
            jnp.int32, (S_LOC, S_GLB), 0)
        col_ids = lax.broadcasted_iota(jnp.int32, (S_LOC, S_GLB), 1)
        mask = (col_ids // BLK) <= (row_ids // BLK)
        neg = jnp.float32(-1e9)

        for b in range(B):
            q_b = jnp.dot(x_ref[b], wq_ref[...],
                          preferred_element_type=jnp.float32)
            ctx_parts = []
            for hh in range(HQ):
                qh = q_b[:, hh * DH:(hh + 1) * DH]
                k_h = kv_ref[:, 0, b, :, hh, :].reshape(S_GLB, DH)
                v_h = kv_ref[:, 1, b, :, hh, :].reshape(S_GLB, DH)
                s = lax.dot_general(
                    qh, k_h, (((1,), (1,)), ((), ())),
                    preferred_element_type=jnp.float32) * SCALE
                s = jnp.where(mask, s, neg)
                m = jnp.max(s, axis=1, keepdims=True)
                w = jnp.exp(s - m)
                w = w / jnp.sum(w, axis=1, keepdims=True)
                ctx_parts.append(
                    jnp.dot(w, v_h, preferred_element_type=jnp.float32))
            ctx_b = jnp.concatenate(ctx_parts, axis=1)
            out_ref[b] = jnp.dot(ctx_b, wo_ref[...],
                                 preferred_element_type=jnp.float32)

    return pl.pallas_call(
        body,
        out_shape=jax.ShapeDtypeStruct((B, S_LOC, D_MODEL), jnp.float32),
        in_specs=[pl.BlockSpec(memory_space=pltpu.VMEM)] * 5,
        out_specs=pl.BlockSpec(memory_space=pltpu.VMEM),
        scratch_shapes=[
            pltpu.VMEM((N_DEV, 2, B, S_LOC, HQ, DH), jnp.float32),
            pltpu.SemaphoreType.DMA((N_DEV - 1,)),
            pltpu.SemaphoreType.DMA((N_DEV - 1,)),
        ],
        compiler_params=pltpu.CompilerParams(
            collective_id=0,
            vmem_limit_bytes=128 * 1024 * 1024,
        ),
    )(x, Wq, K_ext, V_ext, Wo)


# baseline (device time: 362093 ns/iter reference)
import jax
import jax.numpy as jnp
from jax import lax
from jax.experimental import pallas as pl
from jax.experimental.pallas import tpu as pltpu

N_DEV = 8
B = 2
S_LOC = 512
HQ = 8
DH = 64
D_MODEL = 768
D_QK = HQ * DH
BLK = 64
SCALE = 0.125


def kernel(x, Wq, K_ext, V_ext, Wo):
    K2 = K_ext.reshape(B, S_LOC, D_QK)
    V2 = V_ext.reshape(B, S_LOC, D_QK)

    def body(x_ref, wq_ref, k_ref, v_ref, wo_ref, out_ref,
             kvg_ref, q_ref, acc_ref, den_ref, send_sems, recv_sems):
        my = lax.axis_index("i")
        left = (my - 1 + N_DEV) % N_DEV
        right = (my + 1) % N_DEV

        barrier_sem = pltpu.get_barrier_semaphore()
        for nbr in (left, right):
            pl.semaphore_signal(barrier_sem, inc=1, device_id=(nbr,),
                                device_id_type=pl.DeviceIdType.MESH)
        pl.semaphore_wait(barrier_sem, 2)

        kvg_ref[my, 0] = k_ref[...]
        kvg_ref[my, 1] = v_ref[...]

        for b in range(B):
            q_ref[b] = jnp.dot(x_ref[b], wq_ref[...],
                               preferred_element_type=jnp.float32)
        acc_ref[...] = jnp.zeros((B, S_LOC, D_QK), jnp.float32)
        den_ref[...] = jnp.zeros((B, HQ, S_LOC), jnp.float32)

        row_blk = my * (S_LOC // BLK) + lax.broadcasted_iota(
            jnp.int32, (S_LOC, S_LOC), 0) // BLK
        col_blk = lax.broadcasted_iota(jnp.int32, (S_LOC, S_LOC), 1) // BLK
        neg = jnp.float32(-1e9)

        def compute_chunk(s):
            mask = (s * (S_LOC // BLK) + col_blk) <= row_blk
            for b in range(B):
                for hh in range(HQ):
                    sl = slice(hh * DH, (hh + 1) * DH)
                    qh = q_ref[b, :, sl]
                    kh = kvg_ref[s, 0, b, :, sl]
                    vh = kvg_ref[s, 1, b, :, sl]
                    sc = lax.dot_general(
                        qh, kh, (((1,), (1,)), ((), ())),
                        preferred_element_type=jnp.float32) * SCALE
                    w = jnp.exp(jnp.where(mask, sc, neg))
                    acc_ref[b, :, sl] = acc_ref[b, :, sl] + jnp.dot(
                        w, vh, preferred_element_type=jnp.float32)
                    den_ref[b, hh] = den_ref[b, hh] + jnp.sum(w, axis=1)

        for h in range(N_DEV - 1):
            slot = (my - h + N_DEV) % N_DEV
            rdma = pltpu.make_async_remote_copy(
                src_ref=kvg_ref.at[slot],
                dst_ref=kvg_ref.at[slot],
                send_sem=send_sems.at[h],
                recv_sem=recv_sems.at[h],
                device_id=(right,),
                device_id_type=pl.DeviceIdType.MESH,
            )
            rdma.start()
            compute_chunk(slot)
            rdma.wait()
        compute_chunk((my - (N_DEV - 1) + N_DEV) % N_DEV)

        for b in range(B):
            ctx_b = jnp.concatenate(
                [acc_ref[b, :, hh * DH:(hh + 1) * DH]
                 / den_ref[b, hh][:, None] for hh in range(HQ)], axis=1)
            out_ref[b] = jnp.dot(ctx_b, wo_ref[...],
                                 preferred_element_type=jnp.float32)

    return pl.pallas_call(
        body,
        out_shape=jax.ShapeDtypeStruct((B, S_LOC, D_MODEL), jnp.float32),
        in_specs=[pl.BlockSpec(memory_space=pltpu.VMEM)] * 5,
        out_specs=pl.BlockSpec(memory_space=pltpu.VMEM),
        scratch_shapes=[
            pltpu.VMEM((N_DEV, 2, B, S_LOC, D_QK), jnp.float32),
            pltpu.VMEM((B, S_LOC, D_QK), jnp.float32),
            pltpu.VMEM((B, S_LOC, D_QK), jnp.float32),
            pltpu.VMEM((B, HQ, S_LOC), jnp.float32),
            pltpu.SemaphoreType.DMA((N_DEV - 1,)),
            pltpu.SemaphoreType.DMA((N_DEV - 1,)),
        ],
        compiler_params=pltpu.CompilerParams(
            collective_id=0,
            vmem_limit_bytes=64 * 1024 * 1024,
        ),
    )(x, Wq, K2, V2, Wo)


# device time: 166026 ns/iter; 2.1809x vs baseline; 2.1809x over previous
import jax
import jax.numpy as jnp
from jax import lax
from jax.experimental import pallas as pl
from jax.experimental.pallas import tpu as pltpu

N_DEV = 8
B = 2
S_LOC = 512
HQ = 8
DH = 64
D_MODEL = 768
D_QK = HQ * DH
BLK = 64
SCALE = 0.125


def kernel(x, Wq, K_ext, V_ext, Wo):
    K2 = K_ext.reshape(B, S_LOC, D_QK).astype(jnp.bfloat16)
    V2 = V_ext.reshape(B, S_LOC, D_QK).astype(jnp.bfloat16)

    def body(x_ref, wq_ref, k_ref, v_ref, wo_ref, out_ref,
             kvg_ref, q_ref, acc_ref, den_ref, send_sems, recv_sems):
        my = lax.axis_index("i")
        AXES = (1, 3, 4)

        barrier_sem = pltpu.get_barrier_semaphore()
        for m in AXES:
            pl.semaphore_signal(barrier_sem, inc=1, device_id=(my ^ m,),
                                device_id_type=pl.DeviceIdType.MESH)
        pl.semaphore_wait(barrier_sem, 3)

        kvg_ref[my, 0] = k_ref[...]
        kvg_ref[my, 1] = v_ref[...]

        for b in range(B):
            q_ref[b] = jnp.dot(x_ref[b], wq_ref[...],
                               preferred_element_type=jnp.float32)
        acc_ref[...] = jnp.zeros((B, S_LOC, D_QK), jnp.float32)
        den_ref[...] = jnp.zeros((B, HQ, S_LOC), jnp.float32)

        row_blk = my * (S_LOC // BLK) + lax.broadcasted_iota(
            jnp.int32, (S_LOC, S_LOC), 0) // BLK
        col_blk = lax.broadcasted_iota(jnp.int32, (S_LOC, S_LOC), 1) // BLK
        neg = jnp.float32(-1e9)

        def compute_chunk(s):
            mask = (s * (S_LOC // BLK) + col_blk) <= row_blk
            for b in range(B):
                for hh in range(HQ):
                    sl = slice(hh * DH, (hh + 1) * DH)
                    qh = q_ref[b, :, sl]
                    kh = kvg_ref[s, 0, b, :, sl]
                    vh = kvg_ref[s, 1, b, :, sl]
                    sc = lax.dot_general(
                        qh.astype(jnp.bfloat16), kh, (((1,), (1,)), ((), ())),
                        preferred_element_type=jnp.float32) * SCALE
                    w = jnp.exp(jnp.where(mask, sc, neg))
                    acc_ref[b, :, sl] = acc_ref[b, :, sl] + jnp.dot(
                        w.astype(jnp.bfloat16), vh,
                        preferred_element_type=jnp.float32)
                    den_ref[b, hh] = den_ref[b, hh] + jnp.sum(w, axis=1)

        C_PARTS = ((0, 176), (176, 352), (352, 512))
        ORDERS = ((1, 3, 4), (3, 4, 1), (4, 1, 3))

        def held_blocks(part, step):
            m1 = ORDERS[part][0]
            if step == 0:
                return [(my, 1)]
            if step == 1:
                if m1 == 1:
                    return [(my & 6, 2)]
                return [(my, 1), (my ^ m1, 1)]
            m2 = ORDERS[part][1]
            if m1 == 1:
                return [(my & 4, 4)]
            if m1 == 4:
                return [(my & 6, 2), ((my & 6) ^ 4, 2)]
            return [(my, 1), (my ^ 3, 1), (my ^ 4, 1), (my ^ 7, 1)]

        idx = 0
        for t in range(3):
            descs = []
            for p in range(3):
                c0, c1 = C_PARTS[p]
                partner = my ^ ORDERS[p][t]
                for s0, ns in held_blocks(p, t):
                    rdma = pltpu.make_async_remote_copy(
                        src_ref=kvg_ref.at[pl.ds(s0, ns), :, :,
                                           pl.ds(c0, c1 - c0), :],
                        dst_ref=kvg_ref.at[pl.ds(s0, ns), :, :,
                                           pl.ds(c0, c1 - c0), :],
                        send_sem=send_sems.at[idx],
                        recv_sem=recv_sems.at[idx],
                        device_id=(partner,),
                        device_id_type=pl.DeviceIdType.MESH,
                    )
                    rdma.start()
                    descs.append(rdma)
                    idx += 1
            if t == 0:
                compute_chunk(my)
            for rdma in descs:
                rdma.wait()

        def chunk_body(off, carry):
            compute_chunk((my + off) % N_DEV)
            return carry
        lax.fori_loop(1, N_DEV, chunk_body, 0)

        for b in range(B):
            ctx_b = jnp.concatenate(
                [acc_ref[b, :, hh * DH:(hh + 1) * DH]
                 / den_ref[b, hh][:, None] for hh in range(HQ)], axis=1)
            out_ref[b] = jnp.dot(ctx_b, wo_ref[...],
                                 preferred_element_type=jnp.float32)

    return pl.pallas_call(
        body,
        out_shape=jax.ShapeDtypeStruct((B, S_LOC, D_MODEL), jnp.float32),
        in_specs=[pl.BlockSpec(memory_space=pltpu.VMEM)] * 5,
        out_specs=pl.BlockSpec(memory_space=pltpu.VMEM),
        scratch_shapes=[
            pltpu.VMEM((N_DEV, 2, B, S_LOC, D_QK), jnp.bfloat16),
            pltpu.VMEM((B, S_LOC, D_QK), jnp.float32),
            pltpu.VMEM((B, S_LOC, D_QK), jnp.float32),
            pltpu.VMEM((B, HQ, S_LOC), jnp.float32),
            pltpu.SemaphoreType.DMA((15,)),
            pltpu.SemaphoreType.DMA((15,)),
        ],
        compiler_params=pltpu.CompilerParams(
            collective_id=0,
            vmem_limit_bytes=64 * 1024 * 1024,
        ),
    )(x, Wq, K2, V2, Wo)


# device time: 137017 ns/iter; 2.6427x vs baseline; 1.2117x over previous
import jax
import jax.numpy as jnp
from jax import lax
from jax.experimental import pallas as pl
from jax.experimental.pallas import tpu as pltpu

N_DEV = 8
B = 2
S_LOC = 512
HQ = 8
DH = 64
D_MODEL = 768
D_QK = HQ * DH
BLK = 64
SCALE = 0.125

C_PARTS = ((0, 128), (128, 256), (256, 512))
PART_HEADS = ((0, 1), (2, 3), (4, 5, 6, 7))
ORDERS = ((1, 3, 4), (3, 4, 1), (4, 1, 3))
STEP_DELIVERS = (
    ((1,), (3,), (4,)),
    ((2, 3), (4, 7), (1, 5)),
    ((4, 5, 6, 7), (1, 2, 5, 6), (2, 3, 6, 7)),
)


def kernel(x, Wq, K_ext, V_ext, Wo):
    K2 = K_ext.reshape(B, S_LOC, D_QK).astype(jnp.bfloat16)
    V2 = V_ext.reshape(B, S_LOC, D_QK).astype(jnp.bfloat16)

    def body(x_ref, wq_ref, k_ref, v_ref, wo_ref, out_ref,
             kvg_ref, q_ref, acc_ref, den_ref, send_sems, recv_sems):
        my = lax.axis_index("i")

        barrier_sem = pltpu.get_barrier_semaphore()
        for m in (1, 3, 4):
            pl.semaphore_signal(barrier_sem, inc=1, device_id=(my ^ m,),
                                device_id_type=pl.DeviceIdType.MESH)
        pl.semaphore_wait(barrier_sem, 3)

        kvg_ref[my, 0] = k_ref[...]
        kvg_ref[my, 1] = v_ref[...]

        for b in range(B):
            q_ref[b] = jnp.dot(x_ref[b], wq_ref[...],
                               preferred_element_type=jnp.float32
                               ).astype(jnp.bfloat16)
        acc_ref[...] = jnp.zeros((B, S_LOC, D_QK), jnp.float32)
        den_ref[...] = jnp.zeros((B, HQ, S_LOC), jnp.float32)

        row_blk = my * (S_LOC // BLK) + lax.broadcasted_iota(
            jnp.int32, (S_LOC, S_LOC), 0) // BLK
        col_blk = lax.broadcasted_iota(jnp.int32, (S_LOC, S_LOC), 1) // BLK
        neg = jnp.bfloat16(-1e9)
        ones_k = jnp.ones((S_LOC,), jnp.bfloat16)

        def compute_part(p, s):
            mask = (s * (S_LOC // BLK) + col_blk) <= row_blk
            for b in range(B):
                for hh in PART_HEADS[p]:
                    sl = slice(hh * DH, (hh + 1) * DH)
                    qh = q_ref[b, :, sl]
                    kh = kvg_ref[s, 0, b, :, sl]
                    vh = kvg_ref[s, 1, b, :, sl]
                    sc = lax.dot_general(
                        qh, kh, (((1,), (1,)), ((), ())),
                        preferred_element_type=jnp.float32) * SCALE
                    w = jnp.exp(jnp.where(mask, sc.astype(jnp.bfloat16), neg))
                    acc_ref[b, :, sl] = acc_ref[b, :, sl] + jnp.dot(
                        w, vh, preferred_element_type=jnp.float32)
                    den_ref[b, hh] = den_ref[b, hh] + lax.dot_general(
                        w, ones_k, (((1,), (0,)), ((), ())),
                        preferred_element_type=jnp.float32)

        def held_blocks(part, step):
            m1 = ORDERS[part][0]
            if step == 0:
                return [(my, 1)]
            if step == 1:
                if m1 == 1:
                    return [(my & 6, 2)]
                return [(my, 1), (my ^ m1, 1)]
            if m1 == 1:
                return [(my & 4, 4)]
            if m1 == 4:
                return [(my & 6, 2), ((my & 6) ^ 4, 2)]
            return [(my, 1), (my ^ 3, 1), (my ^ 4, 1), (my ^ 7, 1)]

        idx = 0
        for t in range(3):
            descs = []
            for p in range(3):
                c0, c1 = C_PARTS[p]
                partner = my ^ ORDERS[p][t]
                for s0, ns in held_blocks(p, t):
                    rdma = pltpu.make_async_remote_copy(
                        src_ref=kvg_ref.at[pl.ds(s0, ns), :, :, :,
                                           pl.ds(c0, c1 - c0)],
                        dst_ref=kvg_ref.at[pl.ds(s0, ns), :, :, :,
                                           pl.ds(c0, c1 - c0)],
                        send_sem=send_sems.at[idx],
                        recv_sem=recv_sems.at[idx],
                        device_id=(partner,),
                        device_id_type=pl.DeviceIdType.MESH,
                    )
                    rdma.start()
                    descs.append(rdma)
                    idx += 1
            if t == 0:
                for p in range(3):
                    compute_part(p, my)
            else:
                for p in range(3):
                    for g in STEP_DELIVERS[t - 1][p]:
                        compute_part(p, my ^ g)
            for rdma in descs:
                rdma.wait()

        for p in range(3):
            for g in STEP_DELIVERS[2][p]:
                compute_part(p, my ^ g)

        for b in range(B):
            ctx_b = jnp.concatenate(
                [acc_ref[b, :, hh * DH:(hh + 1) * DH]
                 / den_ref[b, hh][:, None] for hh in range(HQ)], axis=1)
            out_ref[b] = jnp.dot(ctx_b, wo_ref[...],
                                 preferred_element_type=jnp.float32)

    return pl.pallas_call(
        body,
        out_shape=jax.ShapeDtypeStruct((B, S_LOC, D_MODEL), jnp.float32),
        in_specs=[pl.BlockSpec(memory_space=pltpu.VMEM)] * 5,
        out_specs=pl.BlockSpec(memory_space=pltpu.VMEM),
        scratch_shapes=[
            pltpu.VMEM((N_DEV, 2, B, S_LOC, D_QK), jnp.bfloat16),
            pltpu.VMEM((B, S_LOC, D_QK), jnp.bfloat16),
            pltpu.VMEM((B, S_LOC, D_QK), jnp.float32),
            pltpu.VMEM((B, HQ, S_LOC), jnp.float32),
            pltpu.SemaphoreType.DMA((15,)),
            pltpu.SemaphoreType.DMA((15,)),
        ],
        compiler_params=pltpu.CompilerParams(
            collective_id=0,
            vmem_limit_bytes=64 * 1024 * 1024,
        ),
    )(x, Wq, K2, V2, Wo)


# device time: 117558 ns/iter; 3.0801x vs baseline; 1.1655x over previous
import jax
import jax.numpy as jnp
from jax import lax
from jax.experimental import pallas as pl
from jax.experimental.pallas import tpu as pltpu

N_DEV = 8
B = 2
S_LOC = 512
HQ = 8
DH = 64
D_MODEL = 768
D_QK = HQ * DH
BLK = 64
SCALE = 0.125

C_PARTS = ((0, 128), (128, 256), (256, 512))
PART_HEADS = ((0, 1), (2, 3), (4, 5, 6, 7))
ORDERS = ((1, 3, 4), (3, 4, 1), (4, 1, 3))


def kernel(x, Wq, K_ext, V_ext, Wo):
    K2 = K_ext.reshape(B, S_LOC, D_QK).astype(jnp.bfloat16)
    V2 = V_ext.reshape(B, S_LOC, D_QK).astype(jnp.bfloat16)

    def body(x_ref, wq_ref, k_ref, v_ref, wo_ref, out_ref,
             kvg_ref, q_ref, acc_ref, den_ref, send_sems, recv_sems):
        my = lax.axis_index("i")

        barrier_sem = pltpu.get_barrier_semaphore()
        for m in (1, 3, 4):
            pl.semaphore_signal(barrier_sem, inc=1, device_id=(my ^ m,),
                                device_id_type=pl.DeviceIdType.MESH)
        pl.semaphore_wait(barrier_sem, 3)

        kvg_ref[my, 0] = k_ref[...]
        kvg_ref[my, 1] = v_ref[...]

        for b in range(B):
            q_ref[b] = jnp.dot(x_ref[b], wq_ref[...],
                               preferred_element_type=jnp.float32
                               ).astype(jnp.bfloat16)
        acc_ref[...] = jnp.zeros((B, S_LOC, D_QK), jnp.float32)
        den_ref[...] = jnp.zeros((B, HQ, S_LOC), jnp.float32)

        row_blk = my * (S_LOC // BLK) + lax.broadcasted_iota(
            jnp.int32, (S_LOC, S_LOC), 0) // BLK
        col_blk = lax.broadcasted_iota(jnp.int32, (S_LOC, S_LOC), 1) // BLK
        neg = jnp.bfloat16(-1e9)
        ones_k = jnp.ones((S_LOC,), jnp.bfloat16)

        def compute_part(p, s):
            mask = (s * (S_LOC // BLK) + col_blk) <= row_blk
            for b in range(B):
                for hh in PART_HEADS[p]:
                    sl = slice(hh * DH, (hh + 1) * DH)
                    qh = q_ref[b, :, sl]
                    kh = kvg_ref[s, 0, b, :, sl]
                    vh = kvg_ref[s, 1, b, :, sl]
                    sc = lax.dot_general(
                        qh, kh, (((1,), (1,)), ((), ())),
                        preferred_element_type=jnp.float32) * SCALE
                    w = jnp.exp(jnp.where(mask, sc.astype(jnp.bfloat16), neg))
                    acc_ref[b, :, sl] = acc_ref[b, :, sl] + jnp.dot(
                        w, vh, preferred_element_type=jnp.float32)
                    den_ref[b, hh] = den_ref[b, hh] + lax.dot_general(
                        w, ones_k, (((1,), (0,)), ((), ())),
                        preferred_element_type=jnp.float32)

        def held_slots(p, t, base):
            m1 = ORDERS[p][0]
            if t == 0:
                return [base]
            if t == 1:
                if m1 == 1:
                    return [base & 6, (base & 6) + 1]
                return [base, base ^ m1]
            if m1 == 1:
                return [base & 4, (base & 4) + 1,
                        (base & 4) + 2, (base & 4) + 3]
            if m1 == 4:
                return [base & 6, (base & 6) + 1,
                        (base & 6) ^ 4, ((base & 6) ^ 4) + 1]
            return [base, base ^ 3, base ^ 4, base ^ 7]

        idx = 0
        for t in range(3):
            rds, recvs = [], []
            for p in range(3):
                c0, c1 = C_PARTS[p]
                partner = my ^ ORDERS[p][t]
                rds.append([])
                recvs.append(held_slots(p, t, partner))
                for s in held_slots(p, t, my):
                    rdma = pltpu.make_async_remote_copy(
                        src_ref=kvg_ref.at[s, :, :, :, pl.ds(c0, c1 - c0)],
                        dst_ref=kvg_ref.at[s, :, :, :, pl.ds(c0, c1 - c0)],
                        send_sem=send_sems.at[idx],
                        recv_sem=recv_sems.at[idx],
                        device_id=(partner,),
                        device_id_type=pl.DeviceIdType.MESH,
                    )
                    rdma.start()
                    rds[p].append(rdma)
                    idx += 1
            if t == 0:
                for p in range(3):
                    compute_part(p, my)
            for r in range(len(rds[0])):
                for p in range(3):
                    rds[p][r].wait()
                    compute_part(p, recvs[p][r])

        for b in range(B):
            ctx_b = jnp.concatenate(
                [acc_ref[b, :, hh * DH:(hh + 1) * DH]
                 / den_ref[b, hh][:, None] for hh in range(HQ)], axis=1)
            out_ref[b] = jnp.dot(ctx_b, wo_ref[...],
                                 preferred_element_type=jnp.float32)

    return pl.pallas_call(
        body,
        out_shape=jax.ShapeDtypeStruct((B, S_LOC, D_MODEL), jnp.float32),
        in_specs=[pl.BlockSpec(memory_space=pltpu.VMEM)] * 5,
        out_specs=pl.BlockSpec(memory_space=pltpu.VMEM),
        scratch_shapes=[
            pltpu.VMEM((N_DEV, 2, B, S_LOC, D_QK), jnp.bfloat16),
            pltpu.VMEM((B, S_LOC, D_QK), jnp.bfloat16),
            pltpu.VMEM((B, S_LOC, D_QK), jnp.float32),
            pltpu.VMEM((B, HQ, S_LOC), jnp.float32),
            pltpu.SemaphoreType.DMA((21,)),
            pltpu.SemaphoreType.DMA((21,)),
        ],
        compiler_params=pltpu.CompilerParams(
            collective_id=0,
            vmem_limit_bytes=64 * 1024 * 1024,
        ),
    )(x, Wq, K2, V2, Wo)


# device time: 112394 ns/iter; 3.2216x vs baseline; 1.0459x over previous
import jax
import jax.numpy as jnp
from jax import lax
from jax.experimental import pallas as pl
from jax.experimental.pallas import tpu as pltpu

N_DEV = 8
B = 2
S_LOC = 512
HQ = 8
DH = 64
D_MODEL = 768
D_QK = HQ * DH
BLK = 64
SCALE = 0.125

C_PARTS = ((0, 128), (128, 256), (256, 512))
PART_HEADS = ((0, 1), (2, 3), (4, 5, 6, 7))
ORDERS = ((1, 3, 4), (3, 4, 1), (4, 1, 3))


def kernel(x, Wq, K_ext, V_ext, Wo):
    K2 = K_ext.reshape(B, S_LOC, D_QK).astype(jnp.bfloat16)
    V2 = V_ext.reshape(B, S_LOC, D_QK).astype(jnp.bfloat16)

    def body(x_ref, wq_ref, k_ref, v_ref, wo_ref, out_ref,
             kvg_ref, q_ref, acc_ref, den_ref, send_sems, recv_sems):
        my = lax.axis_index("i")

        barrier_sem = pltpu.get_barrier_semaphore()
        for m in (1, 3, 4):
            pl.semaphore_signal(barrier_sem, inc=1, device_id=(my ^ m,),
                                device_id_type=pl.DeviceIdType.MESH)
        pl.semaphore_wait(barrier_sem, 3)

        kvg_ref[my, 0] = k_ref[...]
        kvg_ref[my, 1] = v_ref[...]

        for b in range(B):
            q_ref[b] = jnp.dot(x_ref[b], wq_ref[...],
                               preferred_element_type=jnp.float32
                               ).astype(jnp.bfloat16)
        acc_ref[...] = jnp.zeros((B, S_LOC, D_QK), jnp.float32)
        den_ref[...] = jnp.zeros((B, HQ, S_LOC), jnp.float32)

        row_blk = my * (S_LOC // BLK) + lax.broadcasted_iota(
            jnp.int32, (S_LOC, S_LOC), 0) // BLK
        col_blk = lax.broadcasted_iota(jnp.int32, (S_LOC, S_LOC), 1) // BLK
        neg = jnp.bfloat16(-1e9)
        ones_k = jnp.ones((S_LOC,), jnp.bfloat16)

        def compute_part(p, s):
            mask = (s * (S_LOC // BLK) + col_blk) <= row_blk
            for b in range(B):
                for hh in PART_HEADS[p]:
                    sl = slice(hh * DH, (hh + 1) * DH)
                    qh = q_ref[b, :, sl]
                    kh = kvg_ref[s, 0, b, :, sl]
                    vh = kvg_ref[s, 1, b, :, sl]
                    sc = lax.dot_general(
                        qh, kh, (((1,), (1,)), ((), ())),
                        preferred_element_type=jnp.float32) * SCALE
                    w = jnp.exp(jnp.where(mask, sc.astype(jnp.bfloat16), neg))
                    acc_ref[b, :, sl] = acc_ref[b, :, sl] + jnp.dot(
                        w, vh, preferred_element_type=jnp.float32)
                    den_ref[b, hh] = den_ref[b, hh] + lax.dot_general(
                        w, ones_k, (((1,), (0,)), ((), ())),
                        preferred_element_type=jnp.float32)

        def held_slots(p, t, base):
            m1 = ORDERS[p][0]
            if t == 0:
                return [base]
            if t == 1:
                if m1 == 1:
                    return [base & 6, (base & 6) + 1]
                return [base, base ^ m1]
            if m1 == 1:
                return [base & 4, (base & 4) + 1,
                        (base & 4) + 2, (base & 4) + 3]
            if m1 == 4:
                return [base & 6, (base & 6) + 1,
                        (base & 6) ^ 4, ((base & 6) ^ 4) + 1]
            return [base, base ^ 3, base ^ 4, base ^ 7]

        idx = 0
        pending = [(p, my) for p in range(3)]
        for t in range(3):
            rds, recvs = [], []
            for p in range(3):
                c0, c1 = C_PARTS[p]
                partner = my ^ ORDERS[p][t]
                rds.append([])
                recvs.append(held_slots(p, t, partner))
                for s in held_slots(p, t, my):
                    rdma = pltpu.make_async_remote_copy(
                        src_ref=kvg_ref.at[s, :, :, :, pl.ds(c0, c1 - c0)],
                        dst_ref=kvg_ref.at[s, :, :, :, pl.ds(c0, c1 - c0)],
                        send_sem=send_sems.at[idx],
                        recv_sem=recv_sems.at[idx],
                        device_id=(partner,),
                        device_id_type=pl.DeviceIdType.MESH,
                    )
                    rdma.start()
                    rds[p].append(rdma)
                    idx += 1
            for p, s in pending:
                compute_part(p, s)
            n = len(rds[0])
            for r in range(n):
                for p in range(3):
                    rds[p][r].wait()
                if r < n - 1:
                    for p in range(3):
                        compute_part(p, recvs[p][r])
                else:
                    pending = [(p, recvs[p][r]) for p in range(3)]
        for p, s in pending:
            compute_part(p, s)

        for b in range(B):
            ctx_b = jnp.concatenate(
                [acc_ref[b, :, hh * DH:(hh + 1) * DH]
                 / den_ref[b, hh][:, None] for hh in range(HQ)], axis=1)
            out_ref[b] = jnp.dot(ctx_b, wo_ref[...],
                                 preferred_element_type=jnp.float32)

    return pl.pallas_call(
        body,
        out_shape=jax.ShapeDtypeStruct((B, S_LOC, D_MODEL), jnp.float32),
        in_specs=[pl.BlockSpec(memory_space=pltpu.VMEM)] * 5,
        out_specs=pl.BlockSpec(memory_space=pltpu.VMEM),
        scratch_shapes=[
            pltpu.VMEM((N_DEV, 2, B, S_LOC, D_QK), jnp.bfloat16),
            pltpu.VMEM((B, S_LOC, D_QK), jnp.bfloat16),
            pltpu.VMEM((B, S_LOC, D_QK), jnp.float32),
            pltpu.VMEM((B, HQ, S_LOC), jnp.float32),
            pltpu.SemaphoreType.DMA((21,)),
            pltpu.SemaphoreType.DMA((21,)),
        ],
        compiler_params=pltpu.CompilerParams(
            collective_id=0,
            vmem_limit_bytes=64 * 1024 * 1024,
        ),
    )(x, Wq, K2, V2, Wo)
